# Initial kernel scaffold; baseline (speedup 1.0000x reference)
import jax
import jax.numpy as jnp
from jax.experimental import pallas as pl


def kernel(object_ids, shape_table, texture_table):
    raise NotImplementedError("write your pallas kernel here")


# speedup vs baseline: 1.4603x; 1.4603x over previous
import functools

import jax
import jax.numpy as jnp
from jax import lax
from jax.experimental import pallas as pl
from jax.experimental.pallas import tpu as pltpu
from jax.experimental.pallas import tpu_sc as plsc

_N_CODES = 100000
_D = 128
_B = 16384

_info = plsc.get_sparse_core_info()
_NC = _info.num_cores
_NS = _info.num_subcores
_NW = _NC * _NS
_B_PER_W = _B // _NW
_CHUNK = 128
_N_CHUNKS = _B_PER_W // _CHUNK


def _make_kernel():
    mesh = plsc.VectorSubcoreMesh(core_axis_name="c", subcore_axis_name="s")

    @functools.partial(
        pl.kernel,
        mesh=mesh,
        out_type=(
            jax.ShapeDtypeStruct((_B, _D), jnp.float32),
            jax.ShapeDtypeStruct((_B, _D), jnp.float32),
        ),
        scratch_types=[
            pltpu.VMEM((_N_CHUNKS, _CHUNK), jnp.int32),
            pltpu.VMEM((_CHUNK, _D), jnp.float32),
            pltpu.VMEM((_CHUNK, _D), jnp.float32),
            pltpu.SemaphoreType.DMA,
            pltpu.SemaphoreType.DMA,
        ],
    )
    def k(ids_hbm, shape_hbm, tex_hbm, zs_hbm, zt_hbm,
          idx_v, buf_s, buf_t, sem_s, sem_t):
        wid = lax.axis_index("s") * _NC + lax.axis_index("c")
        base = wid * _B_PER_W
        pltpu.sync_copy(ids_hbm.at[wid], idx_v)
        for c in range(_N_CHUNKS):
            idx_c = idx_v.at[c]
            cp_s = pltpu.async_copy(shape_hbm.at[idx_c], buf_s, sem_s)
            cp_t = pltpu.async_copy(tex_hbm.at[idx_c], buf_t, sem_t)
            row0 = base + c * _CHUNK
            cp_s.wait()
            pltpu.sync_copy(buf_s, zs_hbm.at[pl.ds(row0, _CHUNK)])
            cp_t.wait()
            pltpu.sync_copy(buf_t, zt_hbm.at[pl.ds(row0, _CHUNK)])

    return k


_gather2 = _make_kernel()


def kernel(object_ids, shape_table, texture_table):
    ids = object_ids.astype(jnp.int32).reshape(_NW, _N_CHUNKS, _CHUNK)
    z_s, z_t = _gather2(ids, shape_table, texture_table)
    return (z_s, z_t)


# speedup vs baseline: 1.5639x; 1.0709x over previous
import functools

import jax
import jax.numpy as jnp
from jax import lax
from jax.experimental import pallas as pl
from jax.experimental.pallas import tpu as pltpu
from jax.experimental.pallas import tpu_sc as plsc

_N_CODES = 100000
_D = 128
_B = 16384

_info = plsc.get_sparse_core_info()
_NC = _info.num_cores
_NS = _info.num_subcores
_NW = _NC * _NS
_B_PER_W = _B // _NW
_CHUNK = 128
_N_CHUNKS = _B_PER_W // _CHUNK


def _make_kernel():
    mesh = plsc.VectorSubcoreMesh(core_axis_name="c", subcore_axis_name="s")

    @functools.partial(
        pl.kernel,
        mesh=mesh,
        out_type=(
            jax.ShapeDtypeStruct((_B, _D), jnp.float32),
            jax.ShapeDtypeStruct((_B, _D), jnp.float32),
        ),
        scratch_types=[
            pltpu.VMEM((_N_CHUNKS, _CHUNK), jnp.int32),
            pltpu.VMEM((_CHUNK, _D), jnp.float32),
            pltpu.VMEM((_CHUNK, _D), jnp.float32),
            pltpu.VMEM((_CHUNK, _D), jnp.float32),
            pltpu.VMEM((_CHUNK, _D), jnp.float32),
            pltpu.SemaphoreType.DMA,
            pltpu.SemaphoreType.DMA,
            pltpu.SemaphoreType.DMA,
            pltpu.SemaphoreType.DMA,
            pltpu.SemaphoreType.DMA,
            pltpu.SemaphoreType.DMA,
            pltpu.SemaphoreType.DMA,
            pltpu.SemaphoreType.DMA,
        ],
    )
    def k(ids_hbm, shape_hbm, tex_hbm, zs_hbm, zt_hbm,
          idx_v, b0, b1, b2, b3, g0, g1, g2, g3, w0, w1, w2, w3):
        wid = lax.axis_index("s") * _NC + lax.axis_index("c")
        base = wid * _B_PER_W
        bufs = (b0, b1, b2, b3)
        gsem = (g0, g1, g2, g3)
        wsem = (w0, w1, w2, w3)
        pltpu.sync_copy(ids_hbm.at[wid], idx_v)
        tasks = []
        for c in range(_N_CHUNKS):
            tasks.append((shape_hbm, zs_hbm, c))
            tasks.append((tex_hbm, zt_hbm, c))
        nt = len(tasks)
        nbuf = len(bufs)
        gcps = [None] * nt
        wcps = [None] * nt
        for i in range(nbuf):
            tbl, _, c = tasks[i]
            gcps[i] = pltpu.async_copy(tbl.at[idx_v.at[c]], bufs[i], gsem[i])
        for i in range(nt):
            _, out, c = tasks[i]
            b = i % nbuf
            gcps[i].wait()
            wcps[i] = pltpu.async_copy(
                bufs[b], out.at[pl.ds(base + c * _CHUNK, _CHUNK)], wsem[b])
            j = i + nbuf
            if j < nt:
                tbl_j, _, c_j = tasks[j]
                wcps[i].wait()
                gcps[j] = pltpu.async_copy(
                    tbl_j.at[idx_v.at[c_j]], bufs[b], gsem[b])
        for i in range(nt - nbuf, nt):
            wcps[i].wait()

    return k


_gather2 = _make_kernel()


def kernel(object_ids, shape_table, texture_table):
    ids = object_ids.astype(jnp.int32).reshape(_NW, _N_CHUNKS, _CHUNK)
    z_s, z_t = _gather2(ids, shape_table, texture_table)
    return (z_s, z_t)


# speedup vs baseline: 1.5876x; 1.0151x over previous
import functools

import jax
import jax.numpy as jnp
from jax import lax
from jax.experimental import pallas as pl
from jax.experimental.pallas import tpu as pltpu
from jax.experimental.pallas import tpu_sc as plsc

_N_CODES = 100000
_D = 128
_B = 16384

_info = plsc.get_sparse_core_info()
_NC = _info.num_cores
_NS = _info.num_subcores
_NW = _NC * _NS
_B_PER_W = _B // _NW
_CHUNK = 128
_N_CHUNKS = _B_PER_W // _CHUNK


def _make_kernel():
    mesh = plsc.VectorSubcoreMesh(core_axis_name="c", subcore_axis_name="s")

    @functools.partial(
        pl.kernel,
        mesh=mesh,
        out_type=(
            jax.ShapeDtypeStruct((_B, _D), jnp.float32),
            jax.ShapeDtypeStruct((_B, _D), jnp.float32),
        ),
        scratch_types=(
            [pltpu.VMEM((_N_CHUNKS, _CHUNK), jnp.int32)]
            + [pltpu.VMEM((_CHUNK, _D), jnp.float32)] * 7
            + [pltpu.SemaphoreType.DMA] * 14
        ),
    )
    def k(ids_hbm, shape_hbm, tex_hbm, zs_hbm, zt_hbm, idx_v, *scr):
        wid = lax.axis_index("s") * _NC + lax.axis_index("c")
        base = wid * _B_PER_W
        bufs = scr[:7]
        gsem = scr[7:14]
        wsem = scr[14:21]
        pltpu.sync_copy(ids_hbm.at[wid], idx_v)
        tasks = []
        for c in range(_N_CHUNKS):
            tasks.append((shape_hbm, zs_hbm, c))
            tasks.append((tex_hbm, zt_hbm, c))
        nt = len(tasks)
        nbuf = len(bufs)
        gcps = [None] * nt
        wcps = [None] * nt
        for i in range(nbuf):
            tbl, _, c = tasks[i]
            gcps[i] = pltpu.async_copy(tbl.at[idx_v.at[c]], bufs[i], gsem[i])
        for i in range(nt):
            _, out, c = tasks[i]
            b = i % nbuf
            if i >= nbuf:
                tbl, _, c_i = tasks[i]
                wcps[b].wait()
                gcps[i] = pltpu.async_copy(
                    tbl.at[idx_v.at[c_i]], bufs[b], gsem[b])
            gcps[i].wait()
            wcps[i] = pltpu.async_copy(
                bufs[b], out.at[pl.ds(base + c * _CHUNK, _CHUNK)], wsem[b])
        for i in range(nt - nbuf, nt):
            wcps[i].wait()

    return k


_gather2 = _make_kernel()


def kernel(object_ids, shape_table, texture_table):
    ids = object_ids.astype(jnp.int32).reshape(_NW, _N_CHUNKS, _CHUNK)
    z_s, z_t = _gather2(ids, shape_table, texture_table)
    return (z_s, z_t)
